# Initial kernel scaffold; baseline (speedup 1.0000x reference)
import jax
import jax.numpy as jnp
from jax.experimental import pallas as pl


def kernel(scores, boxes):
    raise NotImplementedError("write your pallas kernel here")


# speedup vs baseline: 267.8749x; 267.8749x over previous
import jax
import jax.numpy as jnp
from jax import lax
from jax.experimental import pallas as pl
from jax.experimental.pallas import tpu as pltpu

_N = 5000
_C = 81
_LP = 5120
_REC = 128
_SCORE_T = 0.05
_IOU_T = 0.5


def _nms_body(s80_ref, s0_ref, x1_ref, y1_ref, x2_ref, y2_ref, out_ref):
    s80 = s80_ref[...]
    s0 = s0_ref[0:1, :]
    x1 = x1_ref[...]
    y1 = y1_ref[...]
    x2 = x2_ref[...]
    y2 = y2_ref[...]

    m = jnp.maximum(jnp.max(s80, axis=0, keepdims=True), s0)
    e80 = jnp.exp(s80 - m)
    denom = jnp.sum(e80, axis=0, keepdims=True) + jnp.exp(s0 - m)
    p80 = e80 / denom

    lane = lax.broadcasted_iota(jnp.int32, (80, _LP), 1)
    valid = lane < _N
    cur0 = jnp.where((p80 > _SCORE_T) & valid, p80, -1.0)

    area = (x2 - x1) * (y2 - y1)

    rlane = lax.broadcasted_iota(jnp.int32, (80, _REC), 1)
    rrow = lax.broadcasted_iota(jnp.int32, (80, _REC), 0)
    rneg = jnp.full((80, _REC), -1.0, dtype=jnp.float32)
    rzero = jnp.zeros((80, _REC), dtype=jnp.float32)

    def body(carry):
        cur, k, rs, rx1, ry1, rx2, ry2 = carry
        mx = jnp.max(cur, axis=1, keepdims=True)
        has = mx > 0.0
        candm = (cur == mx) & has
        idx = jnp.min(jnp.where(candm, lane, _LP), axis=1, keepdims=True)
        oh = lane == idx
        ohf = oh.astype(jnp.float32)
        sx1 = jnp.sum(x1 * ohf, axis=1, keepdims=True)
        sy1 = jnp.sum(y1 * ohf, axis=1, keepdims=True)
        sx2 = jnp.sum(x2 * ohf, axis=1, keepdims=True)
        sy2 = jnp.sum(y2 * ohf, axis=1, keepdims=True)
        sarea = (sx2 - sx1) * (sy2 - sy1)
        iw = jnp.clip(jnp.minimum(x2, sx2) - jnp.maximum(x1, sx1), 0.0)
        ih = jnp.clip(jnp.minimum(y2, sy2) - jnp.maximum(y1, sy1), 0.0)
        inter = iw * ih
        iou = inter / (area + sarea - inter + 1e-9)
        supp = (iou > _IOU_T) & has
        newcur = jnp.where(supp | oh, -1.0, cur)

        slot = jnp.minimum(k, _REC - 1)
        at = rlane == slot
        sc_val = jnp.where(has, mx, -1.0)
        rs = jnp.where(at, sc_val, rs)
        rx1 = jnp.where(at, sx1, rx1)
        ry1 = jnp.where(at, sy1, ry1)
        rx2 = jnp.where(at, sx2, rx2)
        ry2 = jnp.where(at, sy2, ry2)
        return newcur, k + 1, rs, rx1, ry1, rx2, ry2

    def cond(carry):
        return jnp.max(carry[0]) > 0.0

    init = (cur0, jnp.int32(0), rneg, rzero, rzero, rzero, rzero)
    _, _, rs, rx1, ry1, rx2, ry2 = lax.while_loop(cond, body, init)

    flat = rrow * _REC + rlane
    sel0 = jnp.where(rlane < 100, rs, -1.0)
    orow = lax.broadcasted_iota(jnp.int32, (104, 8), 0)
    ocol = lax.broadcasted_iota(jnp.int32, (104, 8), 1)

    def sel_body(t, carry):
        sel, acc = carry
        mx = jnp.max(sel)
        good = mx > 0.0
        fidx = jnp.min(jnp.where(sel == mx, flat, 80 * _REC))
        oh = (flat == fidx) & good
        score = jnp.where(good, mx, 0.0)
        vx1 = jnp.sum(jnp.where(oh, rx1, 0.0))
        vy1 = jnp.sum(jnp.where(oh, ry1, 0.0))
        vx2 = jnp.sum(jnp.where(oh, rx2, 0.0))
        vy2 = jnp.sum(jnp.where(oh, ry2, 0.0))
        row = (jnp.where(ocol == 0, score, 0.0)
               + jnp.where(ocol == 1, vx1, 0.0)
               + jnp.where(ocol == 2, vy1, 0.0)
               + jnp.where(ocol == 3, vx2, 0.0)
               + jnp.where(ocol == 4, vy2, 0.0))
        acc = jnp.where(orow == t, row, acc)
        sel = jnp.where(oh, -1.0, sel)
        return sel, acc

    _, acc = lax.fori_loop(
        0, 100, sel_body, (sel0, jnp.zeros((104, 8), jnp.float32)))
    out_ref[...] = acc


def kernel(scores, boxes):
    st = jnp.transpose(scores)
    st = jnp.pad(st, ((0, 0), (0, _LP - _N)), constant_values=-1e30)
    s80 = st[1:]
    s0 = jnp.pad(st[0:1], ((0, 7), (0, 0)), constant_values=-1e30)
    bt = jnp.transpose(boxes.reshape(_N, _C, 4), (1, 2, 0))[1:]
    bt = jnp.pad(bt, ((0, 0), (0, 0), (0, _LP - _N)))
    x1, y1, x2, y2 = bt[:, 0], bt[:, 1], bt[:, 2], bt[:, 3]

    out = pl.pallas_call(
        _nms_body,
        out_shape=jax.ShapeDtypeStruct((104, 8), jnp.float32),
    )(s80, s0, x1, y1, x2, y2)
    return out[:100, :5]


# speedup vs baseline: 722.7852x; 2.6982x over previous
import functools

import jax
import jax.numpy as jnp
from jax import lax
from jax.experimental import pallas as pl
from jax.experimental.pallas import tpu as pltpu
from jax.experimental.pallas import tpu_sc as plsc

_N = 5000
_C = 81
_LP = 5120
_REC = 128
_K = 512
_KBUF = 640
_SCORE_T = 0.05
_IOU_T = 0.5
_NCHUNK = _LP // 16


def _prep_body(s80_ref, s0_ref, cur_ref):
    s80 = s80_ref[...]
    s0 = s0_ref[0:1, :]
    m = jnp.maximum(jnp.max(s80, axis=0, keepdims=True), s0)
    e80 = jnp.exp(s80 - m)
    denom = jnp.sum(e80, axis=0, keepdims=True) + jnp.exp(s0 - m)
    p80 = e80 / denom
    lane = lax.broadcasted_iota(jnp.int32, (80, _LP), 1)
    cur_ref[...] = jnp.where((p80 > _SCORE_T) & (lane < _N), p80, -1.0)


def _sc_compact_body(cur_hbm, x1_hbm, y1_hbm, x2_hbm, y2_hbm,
                     osc_hbm, ox1_hbm, oy1_hbm, ox2_hbm, oy2_hbm, ocnt_hbm,
                     sc_v, x1_v, y1_v, x2_v, y2_v,
                     bsc_v, bx1_v, by1_v, bx2_v, by2_v, cnt_v):
    info = plsc.get_sparse_core_info()
    nc = info.num_cores
    wid = lax.axis_index("s") * nc + lax.axis_index("c")

    lanev = lax.iota(jnp.int32, 16)
    last = jnp.full((16,), 15, jnp.int32)

    def lgather(x, idx):
        return x.at[idx].get(mode="promise_in_bounds")

    def do_class(j):
        pltpu.sync_copy(cur_hbm.at[j], sc_v)
        pltpu.sync_copy(x1_hbm.at[j], x1_v)
        pltpu.sync_copy(y1_hbm.at[j], y1_v)
        pltpu.sync_copy(x2_hbm.at[j], x2_v)
        pltpu.sync_copy(y2_hbm.at[j], y2_v)

        def chunk(i, off):
            v = sc_v[pl.ds(i * 16, 16)]
            msk = v > 0.0
            c = jnp.where(msk, 1, 0).astype(jnp.int32)
            for d in (1, 2, 4, 8):
                sh = lgather(c, jnp.maximum(lanev - d, 0))
                c = c + jnp.where(lanev >= d, sh, 0)
            idx = off + c - 1
            okm = msk & (idx < _KBUF)
            plsc.store_scatter(bsc_v, [idx], v, mask=okm)
            plsc.store_scatter(bx1_v, [idx], x1_v[pl.ds(i * 16, 16)], mask=okm)
            plsc.store_scatter(by1_v, [idx], y1_v[pl.ds(i * 16, 16)], mask=okm)
            plsc.store_scatter(bx2_v, [idx], x2_v[pl.ds(i * 16, 16)], mask=okm)
            plsc.store_scatter(by2_v, [idx], y2_v[pl.ds(i * 16, 16)], mask=okm)
            return off + lgather(c, last)

        count = lax.fori_loop(0, _NCHUNK, chunk, jnp.zeros((16,), jnp.int32))
        cnt_v[...] = count
        pltpu.sync_copy(cnt_v, ocnt_hbm.at[j])
        pltpu.sync_copy(bsc_v, osc_hbm.at[j])
        pltpu.sync_copy(bx1_v, ox1_hbm.at[j])
        pltpu.sync_copy(by1_v, oy1_hbm.at[j])
        pltpu.sync_copy(bx2_v, ox2_hbm.at[j])
        pltpu.sync_copy(by2_v, oy2_hbm.at[j])

    for t in range(3):
        j = wid + 32 * t

        @pl.when(j < 80)
        def _():
            do_class(j)


def _run_nms(cur0, x1, y1, x2, y2, width):
    lane = lax.broadcasted_iota(jnp.int32, (80, width), 1)
    area = (x2 - x1) * (y2 - y1)
    rlane = lax.broadcasted_iota(jnp.int32, (80, _REC), 1)
    rneg = jnp.full((80, _REC), -1.0, dtype=jnp.float32)
    rzero = jnp.zeros((80, _REC), dtype=jnp.float32)

    def body(carry):
        cur, k, rs, rx1, ry1, rx2, ry2 = carry
        mx = jnp.max(cur, axis=1, keepdims=True)
        has = mx > 0.0
        candm = (cur == mx) & has
        idx = jnp.min(jnp.where(candm, lane, width), axis=1, keepdims=True)
        oh = lane == idx
        sx1 = jnp.sum(jnp.where(oh, x1, 0.0), axis=1, keepdims=True)
        sy1 = jnp.sum(jnp.where(oh, y1, 0.0), axis=1, keepdims=True)
        sx2 = jnp.sum(jnp.where(oh, x2, 0.0), axis=1, keepdims=True)
        sy2 = jnp.sum(jnp.where(oh, y2, 0.0), axis=1, keepdims=True)
        sarea = (sx2 - sx1) * (sy2 - sy1)
        iw = jnp.clip(jnp.minimum(x2, sx2) - jnp.maximum(x1, sx1), 0.0)
        ih = jnp.clip(jnp.minimum(y2, sy2) - jnp.maximum(y1, sy1), 0.0)
        inter = iw * ih
        iou = inter / (area + sarea - inter + 1e-9)
        supp = (iou > _IOU_T) & has
        newcur = jnp.where(supp | oh, -1.0, cur)

        slot = jnp.minimum(k, _REC - 1)
        at = rlane == slot
        sc_val = jnp.where(has, mx, -1.0)
        rs = jnp.where(at, sc_val, rs)
        rx1 = jnp.where(at, sx1, rx1)
        ry1 = jnp.where(at, sy1, ry1)
        rx2 = jnp.where(at, sx2, rx2)
        ry2 = jnp.where(at, sy2, ry2)
        return newcur, k + 1, rs, rx1, ry1, rx2, ry2

    def cond(carry):
        return jnp.max(carry[0]) > 0.0

    init = (cur0, jnp.int32(0), rneg, rzero, rzero, rzero, rzero)
    out = lax.while_loop(cond, body, init)
    return out[2:]


def _nms_body(cnt_ref, csc_ref, cx1_ref, cy1_ref, cx2_ref, cy2_ref,
              cur_ref, x1_ref, y1_ref, x2_ref, y2_ref, out_ref):
    counts = cnt_ref[:, 0:1]

    def compact_path(_):
        pos = lax.broadcasted_iota(jnp.int32, (80, _K), 1)
        live = pos < counts
        csc = jnp.where(live, csc_ref[:, :_K], -1.0)
        return _run_nms(csc, cx1_ref[:, :_K], cy1_ref[:, :_K],
                        cx2_ref[:, :_K], cy2_ref[:, :_K], _K)

    def dense_path(_):
        return _run_nms(cur_ref[...], x1_ref[...], y1_ref[...],
                        x2_ref[...], y2_ref[...], _LP)

    rs, rx1, ry1, rx2, ry2 = lax.cond(
        jnp.max(counts) <= _K, compact_path, dense_path, operand=None)

    rlane = lax.broadcasted_iota(jnp.int32, (80, _REC), 1)
    rrow = lax.broadcasted_iota(jnp.int32, (80, _REC), 0)
    flat = rrow * _REC + rlane
    sel0 = jnp.where(rlane < 100, rs, -1.0)
    orow = lax.broadcasted_iota(jnp.int32, (104, 8), 0)
    ocol = lax.broadcasted_iota(jnp.int32, (104, 8), 1)

    def sel_body(t, carry):
        sel, acc = carry
        mx = jnp.max(sel)
        good = mx > 0.0
        fidx = jnp.min(jnp.where(sel == mx, flat, 80 * _REC))
        oh = (flat == fidx) & good
        score = jnp.where(good, mx, 0.0)
        vx1 = jnp.sum(jnp.where(oh, rx1, 0.0))
        vy1 = jnp.sum(jnp.where(oh, ry1, 0.0))
        vx2 = jnp.sum(jnp.where(oh, rx2, 0.0))
        vy2 = jnp.sum(jnp.where(oh, ry2, 0.0))
        row = (jnp.where(ocol == 0, score, 0.0)
               + jnp.where(ocol == 1, vx1, 0.0)
               + jnp.where(ocol == 2, vy1, 0.0)
               + jnp.where(ocol == 3, vx2, 0.0)
               + jnp.where(ocol == 4, vy2, 0.0))
        acc = jnp.where(orow == t, row, acc)
        sel = jnp.where(oh, -1.0, sel)
        return sel, acc

    _, acc = lax.fori_loop(
        0, 100, sel_body, (sel0, jnp.zeros((104, 8), jnp.float32)))
    out_ref[...] = acc


def kernel(scores, boxes):
    st = jnp.transpose(scores)
    st = jnp.pad(st, ((0, 0), (0, _LP - _N)), constant_values=-1e30)
    s80 = st[1:]
    s0 = jnp.pad(st[0:1], ((0, 7), (0, 0)), constant_values=-1e30)
    bt = jnp.transpose(boxes.reshape(_N, _C, 4), (1, 2, 0))[1:]
    bt = jnp.pad(bt, ((0, 0), (0, 0), (0, _LP - _N)))
    x1, y1, x2, y2 = bt[:, 0], bt[:, 1], bt[:, 2], bt[:, 3]

    cur0 = pl.pallas_call(
        _prep_body,
        out_shape=jax.ShapeDtypeStruct((80, _LP), jnp.float32),
    )(s80, s0)

    fbuf = jax.ShapeDtypeStruct((80, _KBUF), jnp.float32)
    sc_compact = functools.partial(
        pl.kernel,
        out_type=[fbuf, fbuf, fbuf, fbuf, fbuf,
                  jax.ShapeDtypeStruct((80, 16), jnp.int32)],
        mesh=plsc.VectorSubcoreMesh(core_axis_name="c", subcore_axis_name="s"),
        compiler_params=pltpu.CompilerParams(needs_layout_passes=False),
        scratch_types=[pltpu.VMEM((_LP,), jnp.float32)] * 5
        + [pltpu.VMEM((_KBUF,), jnp.float32)] * 5
        + [pltpu.VMEM((16,), jnp.int32)],
    )(_sc_compact_body)
    csc, cx1, cy1, cx2, cy2, ccnt = sc_compact(cur0, x1, y1, x2, y2)

    out = pl.pallas_call(
        _nms_body,
        out_shape=jax.ShapeDtypeStruct((104, 8), jnp.float32),
    )(ccnt, csc, cx1, cy1, cx2, cy2, cur0, x1, y1, x2, y2)
    return out[:100, :5]


# speedup vs baseline: 835.4768x; 1.1559x over previous
import functools

import jax
import jax.numpy as jnp
from jax import lax
from jax.experimental import pallas as pl
from jax.experimental.pallas import tpu as pltpu
from jax.experimental.pallas import tpu_sc as plsc

_N = 5000
_C = 81
_LP = 5120
_REC = 128
_K = 256
_KBUF = 384
_SCORE_T = 0.05
_IOU_T = 0.5
_NCHUNK = _LP // 16


def _prep_body(s80_ref, s0_ref, cur_ref):
    s80 = s80_ref[...]
    s0 = s0_ref[0:1, :]
    m = jnp.maximum(jnp.max(s80, axis=0, keepdims=True), s0)
    e80 = jnp.exp(s80 - m)
    denom = jnp.sum(e80, axis=0, keepdims=True) + jnp.exp(s0 - m)
    p80 = e80 / denom
    lane = lax.broadcasted_iota(jnp.int32, (80, _LP), 1)
    cur_ref[...] = jnp.where((p80 > _SCORE_T) & (lane < _N), p80, -1.0)


def _sc_compact_body(cur_hbm, x1_hbm, y1_hbm, x2_hbm, y2_hbm,
                     osc_hbm, ox1_hbm, oy1_hbm, ox2_hbm, oy2_hbm, ocnt_hbm,
                     sc_v, x1_v, y1_v, x2_v, y2_v,
                     bsc_v, bx1_v, by1_v, bx2_v, by2_v, cnt_v):
    info = plsc.get_sparse_core_info()
    nc = info.num_cores
    wid = lax.axis_index("s") * nc + lax.axis_index("c")

    lanev = lax.iota(jnp.int32, 16)
    last = jnp.full((16,), 15, jnp.int32)

    def lgather(x, idx):
        return x.at[idx].get(mode="promise_in_bounds")

    def do_class(j):
        pltpu.sync_copy(cur_hbm.at[j], sc_v)
        pltpu.sync_copy(x1_hbm.at[j], x1_v)
        pltpu.sync_copy(y1_hbm.at[j], y1_v)
        pltpu.sync_copy(x2_hbm.at[j], x2_v)
        pltpu.sync_copy(y2_hbm.at[j], y2_v)

        def chunk(i, off):
            v = sc_v[pl.ds(i * 16, 16)]
            msk = v > 0.0
            c = jnp.where(msk, 1, 0).astype(jnp.int32)
            for d in (1, 2, 4, 8):
                sh = lgather(c, jnp.maximum(lanev - d, 0))
                c = c + jnp.where(lanev >= d, sh, 0)
            idx = off + c - 1
            okm = msk & (idx < _KBUF)
            plsc.store_scatter(bsc_v, [idx], v, mask=okm)
            plsc.store_scatter(bx1_v, [idx], x1_v[pl.ds(i * 16, 16)], mask=okm)
            plsc.store_scatter(by1_v, [idx], y1_v[pl.ds(i * 16, 16)], mask=okm)
            plsc.store_scatter(bx2_v, [idx], x2_v[pl.ds(i * 16, 16)], mask=okm)
            plsc.store_scatter(by2_v, [idx], y2_v[pl.ds(i * 16, 16)], mask=okm)
            return off + lgather(c, last)

        count = lax.fori_loop(0, _NCHUNK, chunk, jnp.zeros((16,), jnp.int32))
        cnt_v[...] = count
        pltpu.sync_copy(cnt_v, ocnt_hbm.at[j])
        pltpu.sync_copy(bsc_v, osc_hbm.at[j])
        pltpu.sync_copy(bx1_v, ox1_hbm.at[j])
        pltpu.sync_copy(by1_v, oy1_hbm.at[j])
        pltpu.sync_copy(bx2_v, ox2_hbm.at[j])
        pltpu.sync_copy(by2_v, oy2_hbm.at[j])

    for t in range(3):
        j = wid + 32 * t

        @pl.when(j < 80)
        def _():
            do_class(j)


def _run_nms(cur0, x1, y1, x2, y2, width):
    lane = lax.broadcasted_iota(jnp.int32, (80, width), 1)
    area = (x2 - x1) * (y2 - y1)
    rlane = lax.broadcasted_iota(jnp.int32, (80, _REC), 1)
    rneg = jnp.full((80, _REC), -1.0, dtype=jnp.float32)
    rzero = jnp.zeros((80, _REC), dtype=jnp.float32)

    def body(carry):
        cur, k, _, rs, rx1, ry1, rx2, ry2 = carry
        mx = jnp.max(cur, axis=1, keepdims=True)
        has = mx > 0.0
        go = jnp.max(mx) > 0.0
        candm = (cur == mx) & has
        idx = jnp.min(jnp.where(candm, lane, width), axis=1, keepdims=True)
        oh = lane == idx
        sx1 = jnp.sum(jnp.where(oh, x1, 0.0), axis=1, keepdims=True)
        sy1 = jnp.sum(jnp.where(oh, y1, 0.0), axis=1, keepdims=True)
        sx2 = jnp.sum(jnp.where(oh, x2, 0.0), axis=1, keepdims=True)
        sy2 = jnp.sum(jnp.where(oh, y2, 0.0), axis=1, keepdims=True)
        sarea = (sx2 - sx1) * (sy2 - sy1)
        iw = jnp.clip(jnp.minimum(x2, sx2) - jnp.maximum(x1, sx1), 0.0)
        ih = jnp.clip(jnp.minimum(y2, sy2) - jnp.maximum(y1, sy1), 0.0)
        inter = iw * ih
        iou = inter / (area + sarea - inter + 1e-9)
        supp = (iou > _IOU_T) & has
        newcur = jnp.where(supp | oh, -1.0, cur)

        slot = jnp.minimum(k, _REC - 1)
        at = rlane == slot
        sc_val = jnp.where(has, mx, -1.0)
        rs = jnp.where(at, sc_val, rs)
        rx1 = jnp.where(at, sx1, rx1)
        ry1 = jnp.where(at, sy1, ry1)
        rx2 = jnp.where(at, sx2, rx2)
        ry2 = jnp.where(at, sy2, ry2)
        return newcur, k + 1, go, rs, rx1, ry1, rx2, ry2

    def cond(carry):
        return carry[2]

    init = (cur0, jnp.int32(0), jnp.bool_(True),
            rneg, rzero, rzero, rzero, rzero)
    out = lax.while_loop(cond, body, init)
    return out[3:]


def _nms_body(cnt_ref, csc_ref, cx1_ref, cy1_ref, cx2_ref, cy2_ref,
              cur_ref, x1_ref, y1_ref, x2_ref, y2_ref, out_ref):
    counts = cnt_ref[:, 0:1]

    def compact_path(_):
        pos = lax.broadcasted_iota(jnp.int32, (80, _K), 1)
        live = pos < counts
        csc = jnp.where(live, csc_ref[:, :_K], -1.0)
        return _run_nms(csc, cx1_ref[:, :_K], cy1_ref[:, :_K],
                        cx2_ref[:, :_K], cy2_ref[:, :_K], _K)

    def dense_path(_):
        return _run_nms(cur_ref[...], x1_ref[...], y1_ref[...],
                        x2_ref[...], y2_ref[...], _LP)

    rs, rx1, ry1, rx2, ry2 = lax.cond(
        jnp.max(counts) <= _K, compact_path, dense_path, operand=None)

    rlane = lax.broadcasted_iota(jnp.int32, (80, _REC), 1)
    rrow = lax.broadcasted_iota(jnp.int32, (80, _REC), 0)
    flat = rrow * _REC + rlane
    sel0 = jnp.where(rlane < 100, rs, -1.0)
    orow = lax.broadcasted_iota(jnp.int32, (104, 8), 0)
    ocol = lax.broadcasted_iota(jnp.int32, (104, 8), 1)

    def sel_body(t, carry):
        sel, acc = carry
        mx = jnp.max(sel)
        good = mx > 0.0
        fidx = jnp.min(jnp.where(sel == mx, flat, 80 * _REC))
        oh = (flat == fidx) & good
        score = jnp.where(good, mx, 0.0)
        vx1 = jnp.sum(jnp.where(oh, rx1, 0.0))
        vy1 = jnp.sum(jnp.where(oh, ry1, 0.0))
        vx2 = jnp.sum(jnp.where(oh, rx2, 0.0))
        vy2 = jnp.sum(jnp.where(oh, ry2, 0.0))
        row = (jnp.where(ocol == 0, score, 0.0)
               + jnp.where(ocol == 1, vx1, 0.0)
               + jnp.where(ocol == 2, vy1, 0.0)
               + jnp.where(ocol == 3, vx2, 0.0)
               + jnp.where(ocol == 4, vy2, 0.0))
        acc = jnp.where(orow == t, row, acc)
        sel = jnp.where(oh, -1.0, sel)
        return sel, acc

    _, acc = lax.fori_loop(
        0, 100, sel_body, (sel0, jnp.zeros((104, 8), jnp.float32)))
    out_ref[...] = acc


def kernel(scores, boxes):
    st = jnp.transpose(scores)
    st = jnp.pad(st, ((0, 0), (0, _LP - _N)), constant_values=-1e30)
    s80 = st[1:]
    s0 = jnp.pad(st[0:1], ((0, 7), (0, 0)), constant_values=-1e30)
    bt = jnp.transpose(boxes.reshape(_N, _C, 4), (1, 2, 0))[1:]
    bt = jnp.pad(bt, ((0, 0), (0, 0), (0, _LP - _N)))
    x1, y1, x2, y2 = bt[:, 0], bt[:, 1], bt[:, 2], bt[:, 3]

    cur0 = pl.pallas_call(
        _prep_body,
        out_shape=jax.ShapeDtypeStruct((80, _LP), jnp.float32),
    )(s80, s0)

    fbuf = jax.ShapeDtypeStruct((80, _KBUF), jnp.float32)
    sc_compact = functools.partial(
        pl.kernel,
        out_type=[fbuf, fbuf, fbuf, fbuf, fbuf,
                  jax.ShapeDtypeStruct((80, 16), jnp.int32)],
        mesh=plsc.VectorSubcoreMesh(core_axis_name="c", subcore_axis_name="s"),
        compiler_params=pltpu.CompilerParams(needs_layout_passes=False),
        scratch_types=[pltpu.VMEM((_LP,), jnp.float32)] * 5
        + [pltpu.VMEM((_KBUF,), jnp.float32)] * 5
        + [pltpu.VMEM((16,), jnp.int32)],
    )(_sc_compact_body)
    csc, cx1, cy1, cx2, cy2, ccnt = sc_compact(cur0, x1, y1, x2, y2)

    out = pl.pallas_call(
        _nms_body,
        out_shape=jax.ShapeDtypeStruct((104, 8), jnp.float32),
    )(ccnt, csc, cx1, cy1, cx2, cy2, cur0, x1, y1, x2, y2)
    return out[:100, :5]
